# baseline (speedup 1.0000x reference)
import functools

import jax
import jax.numpy as jnp
from jax import lax
from jax.experimental import pallas as pl
from jax.experimental.pallas import tpu as pltpu
from jax.experimental.pallas import tpu_sc as plsc

B = 16
S = 4096
D = 1024
LANES = 16
NUM_CORES = 2
NUM_SUBCORES = 16
NUM_WORKERS = NUM_CORES * NUM_SUBCORES
COLS = D // NUM_WORKERS
GROUPS = COLS // LANES
CHUNK = 128


def _avg_body(x_hbm, len_hbm, out_hbm, len_v, buf_v, out_v):
    wid = lax.axis_index("c") * NUM_SUBCORES + lax.axis_index("s")
    col0 = wid * COLS

    pltpu.sync_copy(len_hbm, len_v)
    lengths = len_v[...]
    iota16 = lax.iota(jnp.int32, LANES)

    def batch_body(i, _):
        length = jnp.sum(jnp.where(iota16 == i, lengths, 0))
        nchunks = lax.div(length + (CHUNK - 1), CHUNK)

        def chunk_body(k, accs):
            base = k * CHUNK
            pltpu.sync_copy(
                x_hbm.at[i, pl.ds(base, CHUNK), pl.ds(col0, COLS)], buf_v
            )
            accs = list(accs)
            for r in range(CHUNK):
                valid = (base + r) < length
                for g in range(GROUPS):
                    v = buf_v[r, pl.ds(g * LANES, LANES)]
                    accs[g] = accs[g] + jnp.where(valid, v, 0.0)
            return tuple(accs)

        zero = jnp.zeros((LANES,), jnp.float32)
        accs = lax.fori_loop(0, nchunks, chunk_body, (zero,) * GROUPS)

        len_vec = jnp.full((LANES,), length, jnp.float32)
        for g in range(GROUPS):
            out_v[pl.ds(g * LANES, LANES)] = accs[g] / len_vec
        pltpu.sync_copy(out_v, out_hbm.at[i, pl.ds(col0, COLS)])
        return 0

    lax.fori_loop(0, B, batch_body, 0)


@jax.jit
def kernel(input, length):
    mesh = plsc.VectorSubcoreMesh(core_axis_name="c", subcore_axis_name="s")
    run = pl.kernel(
        _avg_body,
        out_type=jax.ShapeDtypeStruct((B, D), jnp.float32),
        mesh=mesh,
        scratch_types=[
            pltpu.VMEM((LANES,), jnp.int32),
            pltpu.VMEM((CHUNK, COLS), jnp.float32),
            pltpu.VMEM((COLS,), jnp.float32),
        ],
        compiler_params=pltpu.CompilerParams(
            use_tc_tiling_on_sc=False, needs_layout_passes=False
        ),
    )
    return run(input, length.astype(jnp.int32))
